# speedup vs baseline: 151.9488x; 151.9488x over previous
import functools

import jax
import jax.numpy as jnp
from jax import lax
from jax.experimental import pallas as pl
from jax.experimental.pallas import tpu as pltpu
from jax.experimental.pallas import tpu_sc as plsc

_LANES = 16
_CHUNK = 8192


@functools.lru_cache(maxsize=None)
def _build(n, q):
    info = plsc.get_sparse_core_info()
    nc, ns = info.num_cores, info.num_subcores
    nw = nc * ns
    qw = -(-q // nw)
    qw = -(-qw // 8) * 8
    assert qw <= q
    nchunk = -(-qw // _CHUNK)
    mesh = plsc.VectorSubcoreMesh(core_axis_name="c", subcore_axis_name="s")

    @functools.partial(
        pl.kernel,
        mesh=mesh,
        out_type=jax.ShapeDtypeStruct((q,), jnp.float32),
        scratch_types=[
            pltpu.VMEM((_CHUNK,), jnp.int32),
            pltpu.VMEM((_CHUNK,), jnp.int32),
            pltpu.VMEM((_CHUNK,), jnp.float32),
            pltpu.VMEM((_CHUNK,), jnp.float32),
            pltpu.SemaphoreType.DMA,
            pltpu.SemaphoreType.DMA,
        ],
    )
    def k(x_hbm, y_hbm, ii_hbm, jj_hbm, out_hbm, ii_v, jj_v, xv, yv,
          sem_x, sem_y):
        wid = lax.axis_index("s") * nc + lax.axis_index("c")
        base = jnp.minimum(wid * qw, q - qw)
        hi = jnp.minimum(base + qw, q) - _CHUNK

        def chunk_body(c, carry):
            off = jnp.minimum(base + c * _CHUNK, hi)
            off = pl.multiple_of(off, 8)
            pltpu.sync_copy(ii_hbm.at[pl.ds(off, _CHUNK)], ii_v)
            pltpu.sync_copy(jj_hbm.at[pl.ds(off, _CHUNK)], jj_v)
            cx = pltpu.async_copy(x_hbm.at[ii_v], xv, sem_x)
            cy = pltpu.async_copy(y_hbm.at[jj_v], yv, sem_y)
            cx.wait()
            cy.wait()

            def mul_body(i, mcarry):
                s = pl.ds(i * _LANES, _LANES)
                xv[s] = xv[s] * yv[s]
                return mcarry

            lax.fori_loop(0, _CHUNK // _LANES, mul_body, 0, unroll=8)
            pltpu.sync_copy(xv, out_hbm.at[pl.ds(off, _CHUNK)])
            return carry

        lax.fori_loop(0, nchunk, chunk_body, 0)

    return k


def kernel(x, y, idx_i, idx_j):
    n = x.shape[0]
    q = idx_i.shape[0]
    return _build(n, q)(x, y, idx_i, idx_j)
